# baseline (speedup 1.0000x reference)
import functools

import jax
import jax.numpy as jnp
from jax import lax
from jax.experimental import pallas as pl
from jax.experimental.pallas import tpu as pltpu
from jax.experimental.pallas import tpu_sc as plsc


_ROWS = 4
_DEPTH = 6
_PF = 4


def _make_sc_kernel(b, s, d):
    info = plsc.get_sparse_core_info()
    nc, ns, lanes = info.num_cores, info.num_subcores, info.num_lanes
    nw = nc * ns
    rows_w = s // nw
    assert s % nw == 0 and rows_w % _ROWS == 0
    n_slabs = rows_w // _ROWS
    groups = d // lanes
    mesh = plsc.VectorSubcoreMesh(core_axis_name="c", subcore_axis_name="s")

    @functools.partial(
        pl.kernel,
        mesh=mesh,
        out_type=jax.ShapeDtypeStruct((b, s, d), jnp.float32),
        scratch_types=(
            [pltpu.VMEM((_ROWS, d), jnp.float32)
             for _ in range(_DEPTH * (b + 1))]
            + [pltpu.SemaphoreType.DMA for _ in range(2)]
        ),
    )
    def k(x_hbm, pos_hbm, out_hbm, *bufs_and_sems):
        nbuf = _DEPTH * (b + 1)
        slots = [bufs_and_sems[i * (b + 1):(i + 1) * (b + 1)]
                 for i in range(_DEPTH)]
        lsem, ssem = bufs_and_sems[nbuf:]
        wid = lax.axis_index("s") * nc + lax.axis_index("c")
        base = wid * rows_w

        def load_slab(c):
            slot = slots[c % _DEPTH]
            r0 = base + c * _ROWS
            cps = [pltpu.async_copy(pos_hbm.at[pl.ds(r0, _ROWS)], slot[0],
                                    lsem)]
            for bb in range(b):
                cps.append(pltpu.async_copy(
                    x_hbm.at[bb, pl.ds(r0, _ROWS)], slot[1 + bb], lsem))
            return cps

        def store_slab(c):
            slot = slots[c % _DEPTH]
            r0 = base + c * _ROWS
            return [pltpu.async_copy(
                slot[1 + bb], out_hbm.at[bb, pl.ds(r0, _ROWS)], ssem)
                for bb in range(b)]

        loads = {c: load_slab(c) for c in range(min(_PF, n_slabs))}
        stores = {}

        for c in range(n_slabs):
            slot = slots[c % _DEPTH]
            pbuf = slot[0]
            for cp in loads.pop(c):
                cp.wait()
            drain = c - (_DEPTH - _PF)
            if drain in stores:
                for cp in stores.pop(drain):
                    cp.wait()
            if c + _PF < n_slabs:
                loads[c + _PF] = load_slab(c + _PF)

            def row_body(r, _, slot=slot, pbuf=pbuf):
                def add_body(i, _2, r=r, slot=slot, pbuf=pbuf):
                    sl = pl.ds(i * lanes, lanes)
                    p = pbuf[r, sl]
                    for bb in range(b):
                        plsc.addupdate(slot[1 + bb].at[r, sl], p)
                    return _2

                lax.fori_loop(0, groups, add_body, 0, unroll=4)
                return _

            lax.fori_loop(0, _ROWS, row_body, 0)
            stores[c] = store_slab(c)

        for c in sorted(stores):
            for cp in stores.pop(c):
                cp.wait()

    return k


def kernel(x, pos_table):
    b, s, d = x.shape
    k = _make_sc_kernel(b, s, d)
    return k(x, pos_table[:s])


# speedup vs baseline: 1.0140x; 1.0140x over previous
import functools

import jax
import jax.numpy as jnp
from jax import lax
from jax.experimental import pallas as pl
from jax.experimental.pallas import tpu as pltpu
from jax.experimental.pallas import tpu_sc as plsc


_ROWS = 4
_DEPTH = 6
_PF = 4


def _make_sc_kernel(b, s, d):
    info = plsc.get_sparse_core_info()
    nc, ns, lanes = info.num_cores, info.num_subcores, info.num_lanes
    nw = nc * ns
    rows_w = s // nw
    assert s % nw == 0 and rows_w % _ROWS == 0
    n_slabs = rows_w // _ROWS
    groups = d // lanes
    mesh = plsc.VectorSubcoreMesh(core_axis_name="c", subcore_axis_name="s")

    @functools.partial(
        pl.kernel,
        mesh=mesh,
        out_type=jax.ShapeDtypeStruct((b, s, d), jnp.float32),
        scratch_types=(
            [pltpu.VMEM((_ROWS, d), jnp.float32)
             for _ in range(_DEPTH * (b + 1))]
            + [pltpu.SemaphoreType.DMA for _ in range(2)]
        ),
    )
    def k(x_hbm, pos_hbm, out_hbm, *bufs_and_sems):
        nbuf = _DEPTH * (b + 1)
        slots = [bufs_and_sems[i * (b + 1):(i + 1) * (b + 1)]
                 for i in range(_DEPTH)]
        lsem, ssem = bufs_and_sems[nbuf:]
        wid = lax.axis_index("s") * nc + lax.axis_index("c")
        base = wid * rows_w

        def load_slab(c):
            slot = slots[c % _DEPTH]
            r0 = base + c * _ROWS
            cps = [pltpu.async_copy(pos_hbm.at[pl.ds(r0, _ROWS)], slot[0],
                                    lsem)]
            for bb in range(b):
                cps.append(pltpu.async_copy(
                    x_hbm.at[bb, pl.ds(r0, _ROWS)], slot[1 + bb], lsem))
            return cps

        def store_slab(c):
            slot = slots[c % _DEPTH]
            r0 = base + c * _ROWS
            return [pltpu.async_copy(
                slot[1 + bb], out_hbm.at[bb, pl.ds(r0, _ROWS)], ssem)
                for bb in range(b)]

        loads = {c: load_slab(c) for c in range(min(_PF, n_slabs))}
        stores = {}

        for c in range(n_slabs):
            slot = slots[c % _DEPTH]
            pbuf = slot[0]
            for cp in loads.pop(c):
                cp.wait()

            def row_body(r, _, slot=slot, pbuf=pbuf):
                def add_body(i, _2, r=r, slot=slot, pbuf=pbuf):
                    sl = pl.ds(i * lanes, lanes)
                    p = pbuf[r, sl]
                    for bb in range(b):
                        plsc.addupdate(slot[1 + bb].at[r, sl], p)
                    return _2

                lax.fori_loop(0, groups, add_body, 0, unroll=4)
                return _

            lax.fori_loop(0, _ROWS, row_body, 0)
            drain = c - (_DEPTH - _PF)
            if drain in stores:
                for cp in stores.pop(drain):
                    cp.wait()
            if c + _PF < n_slabs:
                loads[c + _PF] = load_slab(c + _PF)
            stores[c] = store_slab(c)

        for c in sorted(stores):
            for cp in stores.pop(c):
                cp.wait()

    return k


def kernel(x, pos_table):
    b, s, d = x.shape
    k = _make_sc_kernel(b, s, d)
    return k(x, pos_table[:s])
